# baseline (speedup 1.0000x reference)
import jax
import jax.numpy as jnp
from jax import lax
from jax.experimental import pallas as pl
from jax.experimental.pallas import tpu as pltpu
from jax.experimental.pallas import tpu_sc as plsc

NC = 2
NS = 16
NW = NC * NS
L = 16

CH = 16
NBUF = 4
PW = 128


def _build(B, D, V1):
    b_per_w = B // NW
    n_chunks = b_per_w // CH
    mesh = plsc.VectorSubcoreMesh(
        core_axis_name="c", subcore_axis_name="s",
        num_cores=NC, num_subcores=NS)

    def body(idx_hbm, t1_hbm, t2_hbm, out_hbm,
             idx_v, q1_v, q2_v, s_v, o_v, tb, outbuf, sems):
        wid = lax.axis_index("s") * NC + lax.axis_index("c")
        base = wid * b_per_w

        pltpu.sync_copy(idx_hbm.at[pl.ds(base, b_per_w)], idx_v)

        for c in range(n_chunks):
            v = idx_v[pl.ds(c * L, L)]
            m = v < V1
            t = jnp.where(m, v, v - V1)
            pair = lax.shift_right_logical(t, 1)
            q1_v[c, :] = jnp.where(m, pair, 0)
            q2_v[c, :] = jnp.where(m, 0, pair)
            s_v[pl.ds(c * L, L)] = jnp.where(m, 0, 1)
            o_v[pl.ds(c * L, L)] = lax.shift_left(
                lax.bitwise_and(t, 1), 6)

        def fire(cc, b):
            pltpu.async_copy(t1_hbm.at[q1_v.at[cc]], tb.at[b, 0],
                             sems.at[b])
            pltpu.async_copy(t2_hbm.at[q2_v.at[cc]], tb.at[b, 1],
                             sems.at[b])

        def drain(cc, b):
            pltpu.make_async_copy(t1_hbm.at[q1_v.at[cc]], tb.at[b, 0],
                                  sems.at[b]).wait()
            pltpu.make_async_copy(t2_hbm.at[q2_v.at[cc]], tb.at[b, 1],
                                  sems.at[b]).wait()

        for b in range(NBUF):
            fire(b, b)

        for cc in range(n_chunks):
            b = cc % NBUF
            drain(cc, b)
            rbase = cc * CH
            sv = s_v[pl.ds(rbase, L)]
            ov = o_v[pl.ds(rbase, L)]
            for i in range(CH):
                s = sv[i]
                o = ov[i]
                for k in range(D // L):
                    outbuf[rbase + i, pl.ds(k * L, L)] = (
                        tb[b, s, i, pl.ds(o + k * L, L)])
            nxt = cc + NBUF
            if nxt < n_chunks:
                fire(nxt, b)
        pltpu.sync_copy(outbuf, out_hbm.at[pl.ds(base, b_per_w)])

    return pl.kernel(
        body,
        out_type=jax.ShapeDtypeStruct((B, D), jnp.float32),
        mesh=mesh,
        compiler_params=pltpu.CompilerParams(use_tc_tiling_on_sc=False),
        scratch_types=[
            pltpu.VMEM((b_per_w,), jnp.int32),
            pltpu.VMEM((n_chunks, CH), jnp.int32),
            pltpu.VMEM((n_chunks, CH), jnp.int32),
            pltpu.VMEM((b_per_w,), jnp.int32),
            pltpu.VMEM((b_per_w,), jnp.int32),
            pltpu.VMEM((NBUF, 2, CH, PW), jnp.float32),
            pltpu.VMEM((b_per_w, D), jnp.float32),
            pltpu.SemaphoreType.DMA((NBUF,)),
        ],
    )


def kernel(indices, table1, table2):
    B = indices.shape[0]
    V1, D = table1.shape
    V2 = table2.shape[0]
    t1 = table1.reshape(V1 * D // PW, PW)
    t2 = table2.reshape(V2 * D // PW, PW)
    return _build(B, D, V1)(indices.astype(jnp.int32), t1, t2)
